# baseline (speedup 1.0000x reference)
import functools

import jax
import jax.numpy as jnp
from jax import lax
from jax.experimental import pallas as pl
from jax.experimental.pallas import tpu as pltpu
from jax.experimental.pallas import tpu_sc as plsc

D = 1024
LANES = 16
NV = D // LANES
EPS = 1e-5
NC = 2
NS = 16
NW = NC * NS


def _hsum(v):
    idx = lax.iota(jnp.int32, LANES)
    for sh in (8, 4, 2, 1):
        v = v + v.at[idx ^ sh].get(mode="promise_in_bounds")
    return v


def _rsqrt(x):
    i = lax.bitcast_convert_type(x, jnp.int32)
    y = lax.bitcast_convert_type(jnp.int32(0x5F3759DF) - (i >> 1), jnp.float32)
    for _ in range(3):
        y = y * (1.5 - 0.5 * x * y * y)
    return y


@functools.partial(jax.jit, static_argnames=())
def kernel(token, emb, ln_weight, ln_bias):
    B = token.shape[0]
    b_per_w = B // NW
    CH = 16
    RB = 16
    n_chunks = b_per_w // CH

    mesh = plsc.VectorSubcoreMesh(core_axis_name="c", subcore_axis_name="s")

    @functools.partial(
        pl.kernel,
        mesh=mesh,
        out_type=jax.ShapeDtypeStruct((B * D,), jnp.float32),
        scratch_types=[
            pltpu.VMEM((b_per_w,), jnp.int32),
            pltpu.VMEM((2, CH, D), jnp.float32),
            pltpu.VMEM((2, CH, D), jnp.float32),
            pltpu.VMEM((D,), jnp.float32),
            pltpu.VMEM((D,), jnp.float32),
            pltpu.SemaphoreType.DMA((2,)),
            pltpu.SemaphoreType.DMA((2,)),
        ],
    )
    def enc(token_hbm, emb_hbm, w_hbm, b_hbm, out_hbm,
            idx_v, rows_v, out_v, w_v, b_v, sem_g, sem_s):
        wid = lax.axis_index("s") * NC + lax.axis_index("c")
        base = wid * b_per_w
        pltpu.sync_copy(token_hbm.at[pl.ds(base, b_per_w)], idx_v)
        pltpu.sync_copy(w_hbm, w_v)
        pltpu.sync_copy(b_hbm, b_v)

        def gather(c, q):
            row0 = pl.multiple_of(c * CH, 8)
            return pltpu.make_async_copy(
                emb_hbm.at[idx_v.at[pl.ds(row0, CH)]], rows_v.at[q],
                sem_g.at[q])

        def store_copies(c, q):
            off = pl.multiple_of((base + c * CH) * D, 8)
            return [
                pltpu.make_async_copy(
                    out_v.at[q, r], out_hbm.at[pl.ds(off + r * D, D)],
                    sem_s.at[q])
                for r in range(CH)
            ]

        gather(0, 0).start()

        def chunk_body(c, _):
            q = lax.rem(c, 2)
            nq = 1 - q

            @pl.when(c + 1 < n_chunks)
            def _():
                gather(c + 1, nq).start()

            gather(c, q).wait()

            @pl.when(c >= 2)
            def _():
                for cp in store_copies(c - 2, q):
                    cp.wait()

            if True:
                r0 = 0
                z = jnp.zeros((LANES,), jnp.float32)
                s = [z] * RB
                ss = [z] * RB
                for j in range(NV):
                    for r in range(RB):
                        jj = (j + r * 4) % NV
                        v = rows_v[q, r0 + r, pl.ds(jj * LANES, LANES)]
                        s[r] = s[r] + v
                        ss[r] = ss[r] + v * v
                rstd = [None] * RB
                shift = [None] * RB
                for r in range(RB):
                    mean = _hsum(s[r]) * (1.0 / D)
                    var = _hsum(ss[r]) * (1.0 / D) - mean * mean
                    rstd[r] = _rsqrt(var + EPS)
                    shift[r] = mean * rstd[r]
                for j in range(NV):
                    sl = pl.ds(j * LANES, LANES)
                    w = w_v[sl]
                    b = b_v[sl]
                    for r in range(RB):
                        v = rows_v[q, r0 + r, sl]
                        out_v[q, r0 + r, sl] = (v * rstd[r] - shift[r]) * w + b

            for cp in store_copies(c, q):
                cp.start()
            return 0

        lax.fori_loop(0, n_chunks, chunk_body, 0)
        for cp in store_copies(n_chunks - 2, lax.rem(n_chunks - 2, 2)):
            cp.wait()
        for cp in store_copies(n_chunks - 1, lax.rem(n_chunks - 1, 2)):
            cp.wait()

    return enc(token.astype(jnp.int32), emb, ln_weight, ln_bias)


# speedup vs baseline: 1.1047x; 1.1047x over previous
import functools

import jax
import jax.numpy as jnp
from jax import lax
from jax.experimental import pallas as pl
from jax.experimental.pallas import tpu as pltpu
from jax.experimental.pallas import tpu_sc as plsc

D = 1024
LANES = 16
NV = D // LANES
EPS = 1e-5
NC = 2
NS = 16
NW = NC * NS


def _hsum(v):
    idx = lax.iota(jnp.int32, LANES)
    for sh in (8, 4, 2, 1):
        v = v + v.at[idx ^ sh].get(mode="promise_in_bounds")
    return v


def _rsqrt(x):
    i = lax.bitcast_convert_type(x, jnp.int32)
    y = lax.bitcast_convert_type(jnp.int32(0x5F3759DF) - (i >> 1), jnp.float32)
    for _ in range(3):
        y = y * (1.5 - 0.5 * x * y * y)
    return y


@functools.partial(jax.jit, static_argnames=())
def kernel(token, emb, ln_weight, ln_bias):
    B = token.shape[0]
    b_per_w = B // NW
    CH = 16
    RB = 16
    n_chunks = b_per_w // CH

    mesh = plsc.VectorSubcoreMesh(core_axis_name="c", subcore_axis_name="s")

    @functools.partial(
        pl.kernel,
        mesh=mesh,
        out_type=jax.ShapeDtypeStruct((B, 8, 128), jnp.float32),
        scratch_types=[
            pltpu.VMEM((b_per_w,), jnp.int32),
            pltpu.VMEM((2, CH, D), jnp.float32),
            pltpu.VMEM((2, CH, D), jnp.float32),
            pltpu.VMEM((D,), jnp.float32),
            pltpu.VMEM((D,), jnp.float32),
            pltpu.SemaphoreType.DMA((2,)),
            pltpu.SemaphoreType.DMA((2,)),
        ],
    )
    def enc(token_hbm, emb_hbm, w_hbm, b_hbm, out_hbm,
            idx_v, rows_v, out_v, w_v, b_v, sem_g, sem_s):
        wid = lax.axis_index("s") * NC + lax.axis_index("c")
        base = wid * b_per_w
        pltpu.sync_copy(token_hbm.at[pl.ds(base, b_per_w)], idx_v)
        pltpu.sync_copy(w_hbm, w_v)
        pltpu.sync_copy(b_hbm, b_v)

        def gather(c, q):
            row0 = pl.multiple_of(c * CH, 8)
            return pltpu.make_async_copy(
                emb_hbm.at[idx_v.at[pl.ds(row0, CH)]], rows_v.at[q],
                sem_g.at[q])

        def store_copies(c, q):
            row0 = pl.multiple_of(c * CH, 8)
            return [
                pltpu.make_async_copy(
                    out_v.at[q, :, pl.ds(sb * 128, 128)],
                    out_hbm.at[pl.ds(base + row0, CH), sb],
                    sem_s.at[q])
                for sb in range(8)
            ]

        gather(0, 0).start()

        def chunk_body(c, _):
            q = lax.rem(c, 2)
            nq = 1 - q

            @pl.when(c + 1 < n_chunks)
            def _():
                gather(c + 1, nq).start()

            gather(c, q).wait()

            @pl.when(c >= 2)
            def _():
                for cp in store_copies(c - 2, q):
                    cp.wait()

            if True:
                r0 = 0
                z = jnp.zeros((LANES,), jnp.float32)
                s = [z] * RB
                ss = [z] * RB
                for j in range(NV):
                    for r in range(RB):
                        jj = (j + r * 4) % NV
                        v = rows_v[q, r0 + r, pl.ds(jj * LANES, LANES)]
                        s[r] = s[r] + v
                        ss[r] = ss[r] + v * v
                rstd = [None] * RB
                shift = [None] * RB
                for r in range(RB):
                    mean = _hsum(s[r]) * (1.0 / D)
                    var = _hsum(ss[r]) * (1.0 / D) - mean * mean
                    rstd[r] = _rsqrt(var + EPS)
                    shift[r] = mean * rstd[r]
                for j in range(NV):
                    sl = pl.ds(j * LANES, LANES)
                    w = w_v[sl]
                    b = b_v[sl]
                    for r in range(RB):
                        v = rows_v[q, r0 + r, sl]
                        out_v[q, r0 + r, sl] = (v * rstd[r] - shift[r]) * w + b

            for cp in store_copies(c, q):
                cp.start()
            return 0

        lax.fori_loop(0, n_chunks, chunk_body, 0)
        for cp in store_copies(n_chunks - 2, lax.rem(n_chunks - 2, 2)):
            cp.wait()
        for cp in store_copies(n_chunks - 1, lax.rem(n_chunks - 1, 2)):
            cp.wait()

    out = enc(token.astype(jnp.int32), emb, ln_weight, ln_bias)
    return out.reshape(-1)


# speedup vs baseline: 1.1472x; 1.0385x over previous
import functools

import jax
import jax.numpy as jnp
from jax import lax
from jax.experimental import pallas as pl
from jax.experimental.pallas import tpu as pltpu
from jax.experimental.pallas import tpu_sc as plsc

D = 1024
LANES = 16
NV = D // LANES
EPS = 1e-5
NC = 2
NS = 16
NW = NC * NS


def _hsum(v):
    idx = lax.iota(jnp.int32, LANES)
    for sh in (8, 4, 2, 1):
        v = v + v.at[idx ^ sh].get(mode="promise_in_bounds")
    return v


def _rsqrt(x):
    i = lax.bitcast_convert_type(x, jnp.int32)
    y = lax.bitcast_convert_type(jnp.int32(0x5F3759DF) - (i >> 1), jnp.float32)
    for _ in range(3):
        y = y * (1.5 - 0.5 * x * y * y)
    return y


@functools.partial(jax.jit, static_argnames=())
def kernel(token, emb, ln_weight, ln_bias):
    B = token.shape[0]
    b_per_w = B // NW
    CH = 16
    RB = 16
    n_chunks = b_per_w // CH

    mesh = plsc.VectorSubcoreMesh(core_axis_name="c", subcore_axis_name="s")

    @functools.partial(
        pl.kernel,
        mesh=mesh,
        out_type=jax.ShapeDtypeStruct((B, 8, 128), jnp.float32),
        scratch_types=[
            pltpu.VMEM((b_per_w,), jnp.int32),
            pltpu.VMEM((2, CH, D), jnp.float32),
            pltpu.VMEM((2, CH, D), jnp.float32),
            pltpu.VMEM((D,), jnp.float32),
            pltpu.VMEM((D,), jnp.float32),
            pltpu.SemaphoreType.DMA((2,)),
            pltpu.SemaphoreType.DMA((2,)),
        ],
    )
    def enc(token_hbm, emb_hbm, w_hbm, b_hbm, out_hbm,
            idx_v, rows_v, out_v, w_v, b_v, sem_g, sem_s):
        wid = lax.axis_index("s") * NC + lax.axis_index("c")
        base = wid * b_per_w
        pltpu.sync_copy(token_hbm.at[pl.ds(base, b_per_w)], idx_v)
        pltpu.sync_copy(w_hbm, w_v)
        pltpu.sync_copy(b_hbm, b_v)

        def gather(c, q):
            row0 = pl.multiple_of(c * CH, 8)
            return pltpu.make_async_copy(
                emb_hbm.at[idx_v.at[pl.ds(row0, CH)]], rows_v.at[q],
                sem_g.at[q])

        def store_copies(c, q):
            row0 = pl.multiple_of(c * CH, 8)
            return [
                pltpu.make_async_copy(
                    out_v.at[q, :, pl.ds(sb * 128, 128)],
                    out_hbm.at[pl.ds(base + row0, CH), sb],
                    sem_s.at[q])
                for sb in range(8)
            ]

        gather(0, 0).start()

        def chunk_body(c, _):
            q = lax.rem(c, 2)
            nq = 1 - q

            @pl.when(c + 1 < n_chunks)
            def _():
                gather(c + 1, nq).start()

            gather(c, q).wait()

            @pl.when(c >= 2)
            def _():
                for cp in store_copies(c - 2, q):
                    cp.wait()

            z = jnp.zeros((LANES,), jnp.float32)
            s = [z] * RB
            ss = [z] * RB
            for j in range(NV):
                for r in range(RB):
                    jj = (j + r * 4) % NV
                    v = rows_v[q, r, pl.ds(jj * LANES, LANES)]
                    s[r] = s[r] + v
                    ss[r] = ss[r] + v * v
            rstd = [None] * RB
            shift = [None] * RB
            for r in range(RB):
                mean = _hsum(s[r]) * (1.0 / D)
                var = _hsum(ss[r]) * (1.0 / D) - mean * mean
                rstd[r] = _rsqrt(var + EPS)
                shift[r] = mean * rstd[r]
            for j in range(NV):
                sl = pl.ds(j * LANES, LANES)
                w = w_v[sl]
                b = b_v[sl]
                for r in range(RB):
                    v = rows_v[q, r, sl]
                    out_v[q, r, sl] = (v * rstd[r] - shift[r]) * w + b

            for cp in store_copies(c, q):
                cp.start()
            return 0

        lax.fori_loop(0, n_chunks, chunk_body, 0)
        for cp in store_copies(n_chunks - 2, lax.rem(n_chunks - 2, 2)):
            cp.wait()
        for cp in store_copies(n_chunks - 1, lax.rem(n_chunks - 1, 2)):
            cp.wait()

    out = enc(token.astype(jnp.int32), emb, ln_weight, ln_bias)
    return out.reshape(-1)
